# baseline (speedup 1.0000x reference)
import functools

import jax
import jax.numpy as jnp
from jax import lax
from jax.experimental import pallas as pl
from jax.experimental.pallas import tpu as pltpu
from jax.experimental.pallas import tpu_sc as plsc

_NUM_CLASSES = 1000000
_FEAT = 64
_BATCH = 16384
_LAMBDA = 0.003

_INFO = plsc.get_sparse_core_info()
_NC, _NS, _L = _INFO.num_cores, _INFO.num_subcores, _INFO.num_lanes
_NW = _NC * _NS
_BPW = _BATCH // _NW
_CHUNK = 128
_NCHUNK = _BPW // _CHUNK
_RB = _BPW // _L


def _body(labels_hbm, embT_hbm, centers_hbm, out_hbm,
          idx_v, rows_v, emb_v, out_v, sem_g, sem_e):
    wid = lax.axis_index("s") * _NC + lax.axis_index("c")
    base = wid * _BPW

    pltpu.sync_copy(labels_hbm.at[pl.ds(base, _BPW)], idx_v)
    emb_cp = pltpu.async_copy(embT_hbm.at[:, pl.ds(base, _BPW)], emb_v, sem_e)
    gathers = [
        pltpu.async_copy(
            centers_hbm.at[idx_v.at[pl.ds(j * _CHUNK, _CHUNK)]],
            rows_v.at[pl.ds(j * _CHUNK, _CHUNK), :],
            sem_g,
        )
        for j in range(_NCHUNK)
    ]
    for g in gathers:
        g.wait()
    emb_cp.wait()

    iota = lax.iota(jnp.int32, _L)

    def step(f, accs):
        fvec = jnp.full((_L,), f, jnp.int32)
        out = list(accs)
        for rb in range(_RB):
            ev = emb_v[f, pl.ds(rb * _L, _L)]
            cv = plsc.load_gather(rows_v, [iota + (rb * _L), fvec])
            d = ev - cv
            out[rb % 4] = out[rb % 4] + d * d
        return tuple(out)

    zero = jnp.zeros((_L,), jnp.float32)
    accs = lax.fori_loop(0, _FEAT, step, (zero,) * 4)
    out_v[...] = (accs[0] + accs[1]) + (accs[2] + accs[3])
    pltpu.sync_copy(out_v, out_hbm.at[wid])


@jax.jit
def _center_loss_partials(labels, embT, centers):
    mesh = plsc.VectorSubcoreMesh(core_axis_name="c", subcore_axis_name="s")
    k = functools.partial(
        pl.kernel,
        mesh=mesh,
        out_type=jax.ShapeDtypeStruct((_NW, _L), jnp.float32),
        scratch_types=[
            pltpu.VMEM((_BPW,), jnp.int32),
            pltpu.VMEM((_BPW, _FEAT), jnp.float32),
            pltpu.VMEM((_FEAT, _BPW), jnp.float32),
            pltpu.VMEM((_L,), jnp.float32),
            pltpu.SemaphoreType.DMA,
            pltpu.SemaphoreType.DMA,
        ],
        compiler_params=pltpu.CompilerParams(
            use_tc_tiling_on_sc=False, needs_layout_passes=False),
    )(_body)
    return k(labels, embT, centers)


def kernel(embeddings, labels, centers):
    partials = _center_loss_partials(
        labels.astype(jnp.int32), embeddings.T, centers)
    return jnp.sum(partials) * (_LAMBDA / (_BATCH * _FEAT))


# speedup vs baseline: 1.0232x; 1.0232x over previous
import functools

import jax
import jax.numpy as jnp
from jax import lax
from jax.experimental import pallas as pl
from jax.experimental.pallas import tpu as pltpu
from jax.experimental.pallas import tpu_sc as plsc

_NUM_CLASSES = 1000000
_FEAT = 64
_BATCH = 16384
_LAMBDA = 0.003

_INFO = plsc.get_sparse_core_info()
_NC, _NS, _L = _INFO.num_cores, _INFO.num_subcores, _INFO.num_lanes
_NW = _NC * _NS
_BPW = _BATCH // _NW
_CHUNK = 128
_NCHUNK = _BPW // _CHUNK
_FVEC = _FEAT // _L


def _body(labels_hbm, emb_hbm, centers_hbm, out_hbm,
          idx_v, rows_v, emb_v, out_v, sem_g, sem_e):
    wid = lax.axis_index("s") * _NC + lax.axis_index("c")
    base = wid * _BPW

    pltpu.sync_copy(labels_hbm.at[pl.ds(base, _BPW)], idx_v)
    emb_cp = pltpu.async_copy(
        emb_hbm.at[pl.ds(base, _BPW), :], emb_v, sem_e)
    gathers = [
        pltpu.async_copy(
            centers_hbm.at[idx_v.at[pl.ds(j * _CHUNK, _CHUNK)]],
            rows_v.at[pl.ds(j * _CHUNK, _CHUNK), :],
            sem_g,
        )
        for j in range(_NCHUNK)
    ]
    for g in gathers:
        g.wait()
    emb_cp.wait()

    def step(r, accs):
        out = []
        for c in range(_FVEC):
            ev = emb_v[r, pl.ds(c * _L, _L)]
            cv = rows_v[r, pl.ds(c * _L, _L)]
            d = ev - cv
            out.append(accs[c] + d * d)
        return tuple(out)

    zero = jnp.zeros((_L,), jnp.float32)
    accs = lax.fori_loop(0, _BPW, step, (zero,) * _FVEC)
    out_v[...] = (accs[0] + accs[1]) + (accs[2] + accs[3])
    pltpu.sync_copy(out_v, out_hbm.at[wid])


@jax.jit
def _center_loss_partials(labels, embeddings, centers):
    mesh = plsc.VectorSubcoreMesh(core_axis_name="c", subcore_axis_name="s")
    k = functools.partial(
        pl.kernel,
        mesh=mesh,
        out_type=jax.ShapeDtypeStruct((_NW, _L), jnp.float32),
        scratch_types=[
            pltpu.VMEM((_BPW,), jnp.int32),
            pltpu.VMEM((_BPW, _FEAT), jnp.float32),
            pltpu.VMEM((_BPW, _FEAT), jnp.float32),
            pltpu.VMEM((_L,), jnp.float32),
            pltpu.SemaphoreType.DMA,
            pltpu.SemaphoreType.DMA,
        ],
        compiler_params=pltpu.CompilerParams(use_tc_tiling_on_sc=False),
    )(_body)
    return k(labels, embeddings, centers)


def kernel(embeddings, labels, centers):
    partials = _center_loss_partials(
        labels.astype(jnp.int32), embeddings, centers)
    return jnp.sum(partials) * (_LAMBDA / (_BATCH * _FEAT))
